# baseline (speedup 1.0000x reference)
import functools

import jax
import jax.numpy as jnp
from jax import lax
from jax.experimental import pallas as pl
from jax.experimental.pallas import tpu as pltpu
from jax.experimental.pallas import tpu_sc as plsc

_N = 10000
_E = 320000
_EPS = 1e-5

_NC = 2
_NS = 16
_NW = _NC * _NS
_CHUNK = _E // _NW
_LANES = 16
_VECS = _CHUNK // _LANES

_BN = 1000
_NB = _N // _BN


def _sc_segsum(cols, ebs, idx_full, zeros):
  mesh = plsc.VectorSubcoreMesh(core_axis_name="c", subcore_axis_name="s")

  @functools.partial(
      pl.kernel,
      mesh=mesh,
      compiler_params=pltpu.CompilerParams(needs_layout_passes=False),
      out_type=jax.ShapeDtypeStruct((_NC, 2 * _N), jnp.float32),
      scratch_types=[
          pltpu.VMEM((2 * _N,), jnp.float32),
          pltpu.VMEM((_CHUNK,), jnp.int32),
          pltpu.VMEM((2 * _CHUNK,), jnp.float32),
          pltpu.VMEM((2 * _CHUNK,), jnp.int32),
          pltpu.VMEM_SHARED((2 * _N,), jnp.float32),
          pltpu.SemaphoreType.DMA,
          pltpu.SemaphoreType.DMA,
          pltpu.SemaphoreType.DMA,
      ],
  )
  def k(cols_hbm, ebs_hbm, idxf_hbm, zeros_hbm, out_hbm,
        cols_v, src_v, upd_v, idx_v, acc_sh, sem1, sem2, sem3):
    c = lax.axis_index("c")
    s = lax.axis_index("s")
    wid = c * _NS + s

    @pl.when(s == 0)
    def _zero():
      pltpu.sync_copy(zeros_hbm, acc_sh)

    c1 = pltpu.async_copy(cols_hbm, cols_v, sem1)
    c2 = pltpu.async_copy(ebs_hbm.at[pl.ds(wid * _CHUNK, _CHUNK)], src_v,
                          sem2)
    c3 = pltpu.async_copy(idxf_hbm.at[pl.ds(wid * 2 * _CHUNK, 2 * _CHUNK)],
                          idx_v, sem3)
    c1.wait()
    c2.wait()

    def body(i, carry):
      sl = pl.ds(i * _LANES, _LANES)
      sv = src_v[sl]
      g0 = plsc.load_gather(cols_v, [sv])
      g1 = plsc.load_gather(cols_v, [sv + _BN])
      upd_v[sl] = g0
      upd_v[pl.ds(_CHUNK + i * _LANES, _LANES)] = g1
      return carry

    lax.fori_loop(0, _VECS, body, 0)
    c3.wait()
    plsc.subcore_barrier()
    pltpu.sync_copy(upd_v, acc_sh.at[idx_v], add=True)
    plsc.subcore_barrier()

    @pl.when(s == 0)
    def _flush():
      pltpu.sync_copy(acc_sh, out_hbm.at[c])

  return k(cols, ebs, idx_full, zeros)


def _aterm(ag_blk, cnt_blk, waT):
  av = ag_blk[0]
  inv = 1.0 / jnp.maximum(cnt_blk[0][0:1, :], 1.0)
  a_s = av * inv
  return lax.dot_general(a_s, waT, (((0,), (0,)), ((), ())),
                         preferred_element_type=jnp.float32)


def _tc_layer(h, ag, cnt, whT, waT, b, g, be, whT2_next):
  din = h.shape[1]
  dout = whT.shape[1]

  def body(h_ref, ag_ref, cnt_ref, whT_ref, waT_ref, b_ref, g_ref, be_ref,
           whT2n_ref, out_ref, hw2_ref, o_scr, s1, s2):
    p = pl.program_id(0)
    i = pl.program_id(1)

    @pl.when(p == 0)
    def _compute():
      @pl.when(i == 0)
      def _init():
        s1[...] = jnp.zeros_like(s1)
        s2[...] = jnp.zeros_like(s2)

      out = jnp.dot(h_ref[...], whT_ref[...],
                    preferred_element_type=jnp.float32)
      out = out + _aterm(ag_ref[...], cnt_ref[...], waT_ref[...]) + b_ref[...]
      o_scr[pl.ds(i * _BN, _BN), :] = out
      s1[...] += jnp.sum(out, axis=0, keepdims=True)
      s2[...] += jnp.sum(out * out, axis=0, keepdims=True)

    @pl.when(p == 1)
    def _normalize():
      o = o_scr[pl.ds(i * _BN, _BN), :]
      mean = s1[...] * (1.0 / _N)
      var = s2[...] * (1.0 / _N) - mean * mean
      scale = g_ref[...] * lax.rsqrt(var + _EPS)
      shift = be_ref[...] - mean * scale
      hn = jnp.maximum(o * scale + shift, 0.0)
      out_ref[...] = hn
      hw2t = lax.dot_general(whT2n_ref[...], hn, (((0,), (1,)), ((), ())),
                             preferred_element_type=jnp.float32)
      hw2_ref[...] = hw2t.reshape(1, 2, _BN)

  return pl.pallas_call(
      body,
      grid=(2, _NB),
      in_specs=[
          pl.BlockSpec((_BN, din), lambda p, i: (jnp.where(p == 0, i, 0), 0)),
          pl.BlockSpec((1, 2, _BN),
                       lambda p, i: (jnp.where(p == 0, i, 0), 0, 0)),
          pl.BlockSpec((1, 2, _BN),
                       lambda p, i: (jnp.where(p == 0, i, 0), 0, 0)),
          pl.BlockSpec((din, dout), lambda p, i: (0, 0)),
          pl.BlockSpec((2, dout), lambda p, i: (0, 0)),
          pl.BlockSpec((1, dout), lambda p, i: (0, 0)),
          pl.BlockSpec((1, dout), lambda p, i: (0, 0)),
          pl.BlockSpec((1, dout), lambda p, i: (0, 0)),
          pl.BlockSpec((dout, 2), lambda p, i: (0, 0)),
      ],
      out_specs=[
          pl.BlockSpec((_BN, dout),
                       lambda p, i: (jnp.where(p == 1, i, 0), 0)),
          pl.BlockSpec((1, 2, _BN),
                       lambda p, i: (jnp.where(p == 1, i, 0), 0, 0)),
      ],
      out_shape=[
          jax.ShapeDtypeStruct((_N, dout), jnp.float32),
          jax.ShapeDtypeStruct((_NB, 2, _BN), jnp.float32),
      ],
      scratch_shapes=[
          pltpu.VMEM((_N, dout), jnp.float32),
          pltpu.VMEM((1, dout), jnp.float32),
          pltpu.VMEM((1, dout), jnp.float32),
      ],
  )(h, ag, cnt, whT, waT, b, g, be, whT2_next)


def _tc_prep(x, whT2, sel2):
  din = x.shape[1]

  def body(x_ref, whT2_ref, sel2_ref, hw2_ref, colsx_ref):
    xb = x_ref[...]
    hw2t = lax.dot_general(whT2_ref[...], xb, (((0,), (1,)), ((), ())),
                           preferred_element_type=jnp.float32)
    hw2_ref[...] = hw2t.reshape(1, 2, _BN)
    cxt = lax.dot_general(sel2_ref[...], xb, (((0,), (1,)), ((), ())),
                          preferred_element_type=jnp.float32)
    colsx_ref[...] = cxt.reshape(1, 2, _BN)

  return pl.pallas_call(
      body,
      grid=(_NB,),
      in_specs=[
          pl.BlockSpec((_BN, din), lambda i: (i, 0)),
          pl.BlockSpec((din, 2), lambda i: (0, 0)),
          pl.BlockSpec((din, 2), lambda i: (0, 0)),
      ],
      out_specs=[
          pl.BlockSpec((1, 2, _BN), lambda i: (i, 0, 0)),
          pl.BlockSpec((1, 2, _BN), lambda i: (i, 0, 0)),
      ],
      out_shape=[
          jax.ShapeDtypeStruct((_NB, 2, _BN), jnp.float32),
          jax.ShapeDtypeStruct((_NB, 2, _BN), jnp.float32),
      ],
  )(x, whT2, sel2)


def _tc_cols2(hw2, ag, cnt, waT2, b2, g2, be2):
  def body(hw2_ref, ag_ref, cnt_ref, waT2_ref, b2_ref, g2_ref, be2_ref,
           cols_ref):
    hw = hw2_ref[...]
    av = ag_ref[...]
    inv = 1.0 / jnp.maximum(cnt_ref[...][:, 0:1, :], 1.0)
    a0 = av[:, 0:1, :] * inv
    a1 = av[:, 1:2, :] * inv
    outs = []
    for c in range(2):
      o = (hw[:, c:c + 1, :] + waT2_ref[0:1, c:c + 1] * a0
           + waT2_ref[1:2, c:c + 1] * a1
           + b2_ref[0:1, c:c + 1]).reshape(_NB, _BN)
      s1 = jnp.sum(jnp.sum(o, axis=0, keepdims=True), axis=1, keepdims=True)
      s2 = jnp.sum(jnp.sum(o * o, axis=0, keepdims=True), axis=1,
                   keepdims=True)
      mean = s1 * (1.0 / _N)
      var = s2 * (1.0 / _N) - mean * mean
      scale = g2_ref[0:1, c:c + 1] * lax.rsqrt(var + _EPS)
      shift = be2_ref[0:1, c:c + 1] - mean * scale
      outs.append(jnp.maximum(o * scale + shift, 0.0).reshape(_NB, 1, _BN))
    cols_ref[...] = jnp.concatenate(outs, axis=1)

  return pl.pallas_call(
      body,
      out_shape=jax.ShapeDtypeStruct((_NB, 2, _BN), jnp.float32),
  )(hw2, ag, cnt, waT2, b2, g2, be2)


def _tc_final(h, ag, cnt, whT, waT, b):
  din = h.shape[1]
  dout = whT.shape[1]

  def body(h_ref, ag_ref, cnt_ref, whT_ref, waT_ref, b_ref, out_ref):
    out = jnp.dot(h_ref[...], whT_ref[...], preferred_element_type=jnp.float32)
    out_ref[...] = (out + _aterm(ag_ref[...], cnt_ref[...], waT_ref[...])
                    + b_ref[...])

  return pl.pallas_call(
      body,
      grid=(_NB,),
      in_specs=[
          pl.BlockSpec((_BN, din), lambda i: (i, 0)),
          pl.BlockSpec((1, 2, _BN), lambda i: (i, 0, 0)),
          pl.BlockSpec((1, 2, _BN), lambda i: (i, 0, 0)),
          pl.BlockSpec((din, dout), lambda i: (0, 0)),
          pl.BlockSpec((2, dout), lambda i: (0, 0)),
          pl.BlockSpec((1, dout), lambda i: (0, 0)),
      ],
      out_specs=pl.BlockSpec((_BN, dout), lambda i: (i, 0)),
      out_shape=jax.ShapeDtypeStruct((_N, dout), jnp.float32),
  )(h, ag, cnt, whT, waT, b)


def kernel(x, edge_index, W1, b1, W2, b2, W3, b3, W4, b4, W5, b5, W6, b6,
           g1, be1, g2, be2, g3, be3, g4, be4, g5, be5):
  zeros = jnp.zeros((2 * _N,), jnp.float32)
  ones = jnp.ones((2 * _N,), jnp.float32)

  ei = jnp.ravel(edge_index)
  eb = ei + (ei // _BN) * _BN
  ebs = eb[:_E]
  ebd = eb[_E:].reshape(_NW, _CHUNK)
  idx_full = jnp.concatenate([ebd, ebd + _BN], axis=1).reshape(-1)

  cpp = _sc_segsum(ones, ebs, idx_full, zeros)
  cnt = (cpp[0] + cpp[1]).reshape(_NB, 2, _BN)

  din0 = x.shape[1]
  sel2 = jnp.eye(din0, 2, dtype=jnp.float32)
  hw2, colsx = _tc_prep(x, W1[0:2, :din0].T, sel2)
  cols = colsx.reshape(-1)

  h = x
  layers = ((W1, b1, g1, be1), (W2, b2, g2, be2), (W3, b3, g3, be3),
            (W4, b4, g4, be4), (W5, b5, g5, be5))
  for li, (W, b, g, be) in enumerate(layers):
    din = h.shape[1]
    W_next = layers[li + 1][0] if li + 1 < len(layers) else W6
    sp = _sc_segsum(cols, ebs, idx_full, zeros)
    ag = (sp[0] + sp[1]).reshape(_NB, 2, _BN)
    cols = _tc_cols2(hw2, ag, cnt, W[0:2, din:].T,
                     b[0:2].reshape(1, -1), g[0:2].reshape(1, -1),
                     be[0:2].reshape(1, -1)).reshape(-1)
    h, hw2 = _tc_layer(h, ag, cnt, W[:, :din].T, W[:, din:].T,
                       b.reshape(1, -1), g.reshape(1, -1),
                       be.reshape(1, -1), W_next[0:2, :256].T)

  din = h.shape[1]
  sp = _sc_segsum(cols, ebs, idx_full, zeros)
  ag = (sp[0] + sp[1]).reshape(_NB, 2, _BN)
  return _tc_final(h, ag, cnt, W6[:, :din].T, W6[:, din:].T,
                   b6.reshape(1, -1))


# speedup vs baseline: 1.0389x; 1.0389x over previous
import functools

import jax
import jax.numpy as jnp
from jax import lax
from jax.experimental import pallas as pl
from jax.experimental.pallas import tpu as pltpu
from jax.experimental.pallas import tpu_sc as plsc

_N = 10000
_E = 320000
_EPS = 1e-5

_NC = 2
_NS = 16
_NW = _NC * _NS
_CHUNK = _E // _NW
_LANES = 16
_VECS = _CHUNK // _LANES

_BN = 1000
_NB = _N // _BN


def _sc_segsum(cols, eb, zeros):
  mesh = plsc.VectorSubcoreMesh(core_axis_name="c", subcore_axis_name="s")

  @functools.partial(
      pl.kernel,
      mesh=mesh,
      compiler_params=pltpu.CompilerParams(needs_layout_passes=False),
      out_type=jax.ShapeDtypeStruct((_NC, 2 * _N), jnp.float32),
      scratch_types=[
          pltpu.VMEM((2 * _N,), jnp.float32),
          pltpu.VMEM((_CHUNK,), jnp.int32),
          pltpu.VMEM((_CHUNK,), jnp.int32),
          pltpu.VMEM((2 * _CHUNK,), jnp.float32),
          pltpu.VMEM((2 * _CHUNK,), jnp.int32),
          pltpu.VMEM_SHARED((2 * _N,), jnp.float32),
          pltpu.SemaphoreType.DMA,
          pltpu.SemaphoreType.DMA,
          pltpu.SemaphoreType.DMA,
      ],
  )
  def k(cols_hbm, eb_hbm, zeros_hbm, out_hbm,
        cols_v, src_v, dst_v, upd_v, idx_v, acc_sh, sem1, sem2, sem3):
    c = lax.axis_index("c")
    s = lax.axis_index("s")
    wid = c * _NS + s
    off = wid * _CHUNK

    @pl.when(s == 0)
    def _zero():
      pltpu.sync_copy(zeros_hbm, acc_sh)

    c1 = pltpu.async_copy(cols_hbm, cols_v, sem1)
    c2 = pltpu.async_copy(eb_hbm.at[pl.ds(off, _CHUNK)], src_v, sem2)
    c3 = pltpu.async_copy(eb_hbm.at[pl.ds(_E + off, _CHUNK)], dst_v, sem3)
    c1.wait()
    c2.wait()
    c3.wait()

    def body(i, carry):
      sl = pl.ds(i * _LANES, _LANES)
      sv = src_v[sl]
      dv = dst_v[sl]
      g0 = plsc.load_gather(cols_v, [sv])
      g1 = plsc.load_gather(cols_v, [sv + _BN])
      upd_v[sl] = g0
      upd_v[pl.ds(_CHUNK + i * _LANES, _LANES)] = g1
      idx_v[sl] = dv
      idx_v[pl.ds(_CHUNK + i * _LANES, _LANES)] = dv + _BN
      return carry

    lax.fori_loop(0, _VECS, body, 0)
    plsc.subcore_barrier()
    pltpu.sync_copy(upd_v, acc_sh.at[idx_v], add=True)
    plsc.subcore_barrier()

    @pl.when(s == 0)
    def _flush():
      pltpu.sync_copy(acc_sh, out_hbm.at[c])

  return k(cols, eb, zeros)


def _aterm(ag_blk, cnt_blk, waT):
  av = ag_blk[0]
  inv = 1.0 / jnp.maximum(cnt_blk[0][0:1, :], 1.0)
  a_s = av * inv
  return lax.dot_general(a_s, waT, (((0,), (0,)), ((), ())),
                         preferred_element_type=jnp.float32)


def _tc_layer(h, ag, cnt, whT, waT, b, g, be, whT2_next):
  din = h.shape[1]
  dout = whT.shape[1]

  def body(h_ref, ag_ref, cnt_ref, whT_ref, waT_ref, b_ref, g_ref, be_ref,
           whT2n_ref, out_ref, hw2_ref, o_scr, s1, s2):
    p = pl.program_id(0)
    i = pl.program_id(1)

    @pl.when(p == 0)
    def _compute():
      @pl.when(i == 0)
      def _init():
        s1[...] = jnp.zeros_like(s1)
        s2[...] = jnp.zeros_like(s2)

      out = jnp.dot(h_ref[...], whT_ref[...],
                    preferred_element_type=jnp.float32)
      out = out + _aterm(ag_ref[...], cnt_ref[...], waT_ref[...]) + b_ref[...]
      o_scr[pl.ds(i * _BN, _BN), :] = out
      s1[...] += jnp.sum(out, axis=0, keepdims=True)
      s2[...] += jnp.sum(out * out, axis=0, keepdims=True)

    @pl.when(p == 1)
    def _normalize():
      o = o_scr[pl.ds(i * _BN, _BN), :]
      mean = s1[...] * (1.0 / _N)
      var = s2[...] * (1.0 / _N) - mean * mean
      scale = g_ref[...] * lax.rsqrt(var + _EPS)
      shift = be_ref[...] - mean * scale
      hn = jnp.maximum(o * scale + shift, 0.0)
      out_ref[...] = hn
      hw2t = lax.dot_general(whT2n_ref[...], hn, (((0,), (1,)), ((), ())),
                             preferred_element_type=jnp.float32)
      hw2_ref[...] = hw2t.reshape(1, 2, _BN)

  return pl.pallas_call(
      body,
      grid=(2, _NB),
      in_specs=[
          pl.BlockSpec((_BN, din), lambda p, i: (jnp.where(p == 0, i, 0), 0)),
          pl.BlockSpec((1, 2, _BN),
                       lambda p, i: (jnp.where(p == 0, i, 0), 0, 0)),
          pl.BlockSpec((1, 2, _BN),
                       lambda p, i: (jnp.where(p == 0, i, 0), 0, 0)),
          pl.BlockSpec((din, dout), lambda p, i: (0, 0)),
          pl.BlockSpec((2, dout), lambda p, i: (0, 0)),
          pl.BlockSpec((1, dout), lambda p, i: (0, 0)),
          pl.BlockSpec((1, dout), lambda p, i: (0, 0)),
          pl.BlockSpec((1, dout), lambda p, i: (0, 0)),
          pl.BlockSpec((dout, 2), lambda p, i: (0, 0)),
      ],
      out_specs=[
          pl.BlockSpec((_BN, dout),
                       lambda p, i: (jnp.where(p == 1, i, 0), 0)),
          pl.BlockSpec((1, 2, _BN),
                       lambda p, i: (jnp.where(p == 1, i, 0), 0, 0)),
      ],
      out_shape=[
          jax.ShapeDtypeStruct((_N, dout), jnp.float32),
          jax.ShapeDtypeStruct((_NB, 2, _BN), jnp.float32),
      ],
      scratch_shapes=[
          pltpu.VMEM((_N, dout), jnp.float32),
          pltpu.VMEM((1, dout), jnp.float32),
          pltpu.VMEM((1, dout), jnp.float32),
      ],
  )(h, ag, cnt, whT, waT, b, g, be, whT2_next)


def _tc_prep(x, whT2, sel2):
  din = x.shape[1]

  def body(x_ref, whT2_ref, sel2_ref, hw2_ref, colsx_ref):
    xb = x_ref[...]
    hw2t = lax.dot_general(whT2_ref[...], xb, (((0,), (1,)), ((), ())),
                           preferred_element_type=jnp.float32)
    hw2_ref[...] = hw2t.reshape(1, 2, _BN)
    cxt = lax.dot_general(sel2_ref[...], xb, (((0,), (1,)), ((), ())),
                          preferred_element_type=jnp.float32)
    colsx_ref[...] = cxt.reshape(1, 2, _BN)

  return pl.pallas_call(
      body,
      grid=(_NB,),
      in_specs=[
          pl.BlockSpec((_BN, din), lambda i: (i, 0)),
          pl.BlockSpec((din, 2), lambda i: (0, 0)),
          pl.BlockSpec((din, 2), lambda i: (0, 0)),
      ],
      out_specs=[
          pl.BlockSpec((1, 2, _BN), lambda i: (i, 0, 0)),
          pl.BlockSpec((1, 2, _BN), lambda i: (i, 0, 0)),
      ],
      out_shape=[
          jax.ShapeDtypeStruct((_NB, 2, _BN), jnp.float32),
          jax.ShapeDtypeStruct((_NB, 2, _BN), jnp.float32),
      ],
  )(x, whT2, sel2)


def _tc_cols2(hw2, ag, cnt, waT2, b2, g2, be2):
  def body(hw2_ref, ag_ref, cnt_ref, waT2_ref, b2_ref, g2_ref, be2_ref,
           cols_ref):
    hw = hw2_ref[...]
    av = ag_ref[...]
    inv = 1.0 / jnp.maximum(cnt_ref[...][:, 0:1, :], 1.0)
    a0 = av[:, 0:1, :] * inv
    a1 = av[:, 1:2, :] * inv
    outs = []
    for c in range(2):
      o = (hw[:, c:c + 1, :] + waT2_ref[0:1, c:c + 1] * a0
           + waT2_ref[1:2, c:c + 1] * a1
           + b2_ref[0:1, c:c + 1]).reshape(_NB, _BN)
      s1 = jnp.sum(jnp.sum(o, axis=0, keepdims=True), axis=1, keepdims=True)
      s2 = jnp.sum(jnp.sum(o * o, axis=0, keepdims=True), axis=1,
                   keepdims=True)
      mean = s1 * (1.0 / _N)
      var = s2 * (1.0 / _N) - mean * mean
      scale = g2_ref[0:1, c:c + 1] * lax.rsqrt(var + _EPS)
      shift = be2_ref[0:1, c:c + 1] - mean * scale
      outs.append(jnp.maximum(o * scale + shift, 0.0).reshape(_NB, 1, _BN))
    cols_ref[...] = jnp.concatenate(outs, axis=1)

  return pl.pallas_call(
      body,
      out_shape=jax.ShapeDtypeStruct((_NB, 2, _BN), jnp.float32),
  )(hw2, ag, cnt, waT2, b2, g2, be2)


def _tc_final(h, ag, cnt, whT, waT, b):
  din = h.shape[1]
  dout = whT.shape[1]

  def body(h_ref, ag_ref, cnt_ref, whT_ref, waT_ref, b_ref, out_ref):
    out = jnp.dot(h_ref[...], whT_ref[...], preferred_element_type=jnp.float32)
    out_ref[...] = (out + _aterm(ag_ref[...], cnt_ref[...], waT_ref[...])
                    + b_ref[...])

  return pl.pallas_call(
      body,
      grid=(_NB,),
      in_specs=[
          pl.BlockSpec((_BN, din), lambda i: (i, 0)),
          pl.BlockSpec((1, 2, _BN), lambda i: (i, 0, 0)),
          pl.BlockSpec((1, 2, _BN), lambda i: (i, 0, 0)),
          pl.BlockSpec((din, dout), lambda i: (0, 0)),
          pl.BlockSpec((2, dout), lambda i: (0, 0)),
          pl.BlockSpec((1, dout), lambda i: (0, 0)),
      ],
      out_specs=pl.BlockSpec((_BN, dout), lambda i: (i, 0)),
      out_shape=jax.ShapeDtypeStruct((_N, dout), jnp.float32),
  )(h, ag, cnt, whT, waT, b)


def kernel(x, edge_index, W1, b1, W2, b2, W3, b3, W4, b4, W5, b5, W6, b6,
           g1, be1, g2, be2, g3, be3, g4, be4, g5, be5):
  zeros = jnp.zeros((2 * _N,), jnp.float32)
  ones = jnp.ones((2 * _N,), jnp.float32)

  ei = jnp.ravel(edge_index)
  eb = ei + (ei // _BN) * _BN

  cpp = _sc_segsum(ones, eb, zeros)
  cnt = (cpp[0] + cpp[1]).reshape(_NB, 2, _BN)

  din0 = x.shape[1]
  sel2 = jnp.eye(din0, 2, dtype=jnp.float32)
  hw2, colsx = _tc_prep(x, W1[0:2, :din0].T, sel2)
  cols = colsx.reshape(-1)

  h = x
  layers = ((W1, b1, g1, be1), (W2, b2, g2, be2), (W3, b3, g3, be3),
            (W4, b4, g4, be4), (W5, b5, g5, be5))
  for li, (W, b, g, be) in enumerate(layers):
    din = h.shape[1]
    W_next = layers[li + 1][0] if li + 1 < len(layers) else W6
    sp = _sc_segsum(cols, eb, zeros)
    ag = (sp[0] + sp[1]).reshape(_NB, 2, _BN)
    cols = _tc_cols2(hw2, ag, cnt, W[0:2, din:].T,
                     b[0:2].reshape(1, -1), g[0:2].reshape(1, -1),
                     be[0:2].reshape(1, -1)).reshape(-1)
    h, hw2 = _tc_layer(h, ag, cnt, W[:, :din].T, W[:, din:].T,
                       b.reshape(1, -1), g.reshape(1, -1),
                       be.reshape(1, -1), W_next[0:2, :256].T)

  din = h.shape[1]
  sp = _sc_segsum(cols, eb, zeros)
  ag = (sp[0] + sp[1]).reshape(_NB, 2, _BN)
  return _tc_final(h, ag, cnt, W6[:, :din].T, W6[:, din:].T,
                   b6.reshape(1, -1))
